# baseline (speedup 1.0000x reference)
import functools

import jax
import jax.numpy as jnp
from jax import lax
from jax.experimental import pallas as pl
from jax.experimental.pallas import tpu as pltpu
from jax.experimental.pallas import tpu_sc as plsc

NN = 10000
EE = 160000
DD = 256
CC = 64
SINK = NN
NPAD = 10112
ZR = NPAD // 16
CR = 632
CRL = NN - 15 * CR
NW = 32
CB = 128
CHUNKS = 40
CH0 = 56
CH1 = 24
EPAD = NW * CHUNKS * CB
ROWS = EPAD // CB
BN = 1000
GRID = NN // BN

_mesh = plsc.VectorSubcoreMesh(core_axis_name="c", subcore_axis_name="s")


@functools.partial(
    pl.kernel,
    out_type=jax.ShapeDtypeStruct((2, 2, NN, 16), jnp.float32),
    mesh=_mesh,
    scratch_types=[
        pltpu.VMEM((CHUNKS, CB), jnp.int32),
        pltpu.VMEM((CHUNKS, CB), jnp.int32),
        pltpu.VMEM((CB, 16), jnp.float32),
        pltpu.VMEM_SHARED((NPAD, 16), jnp.float32),
        pltpu.VMEM_SHARED((NPAD, 16), jnp.float32),
    ],
    compiler_params=pltpu.CompilerParams(use_tc_tiling_on_sc=False),
)
def _deg_kernel(src_hbm, dst_hbm, ones_hbm, zeros_hbm, out_hbm,
                src_v, dst_v, ones_v, acc_in, acc_out):
    c = lax.axis_index("c")
    s = lax.axis_index("s")
    wid = c * 16 + s
    pltpu.sync_copy(ones_hbm, ones_v)
    pltpu.sync_copy(zeros_hbm.at[pl.ds(s * ZR, ZR)], acc_in.at[pl.ds(s * ZR, ZR)])
    pltpu.sync_copy(zeros_hbm.at[pl.ds(s * ZR, ZR)], acc_out.at[pl.ds(s * ZR, ZR)])
    pltpu.sync_copy(src_hbm.at[pl.ds(wid * CHUNKS, CHUNKS)], src_v)
    pltpu.sync_copy(dst_hbm.at[pl.ds(wid * CHUNKS, CHUNKS)], dst_v)
    plsc.subcore_barrier()

    def body(j, carry):
        pltpu.sync_copy(ones_v, acc_in.at[dst_v.at[j]], add=True)
        pltpu.sync_copy(ones_v, acc_out.at[src_v.at[j]], add=True)
        return carry

    lax.fori_loop(0, CHUNKS, body, 0)
    plsc.subcore_barrier()

    @pl.when(s < 15)
    def _():
        pltpu.sync_copy(acc_in.at[pl.ds(s * CR, CR)],
                        out_hbm.at[c, 0, pl.ds(s * CR, CR)])
        pltpu.sync_copy(acc_out.at[pl.ds(s * CR, CR)],
                        out_hbm.at[c, 1, pl.ds(s * CR, CR)])

    @pl.when(s == 15)
    def _():
        pltpu.sync_copy(acc_in.at[pl.ds(15 * CR, CRL)],
                        out_hbm.at[c, 0, pl.ds(15 * CR, CRL)])
        pltpu.sync_copy(acc_out.at[pl.ds(15 * CR, CRL)],
                        out_hbm.at[c, 1, pl.ds(15 * CR, CRL)])


@functools.partial(
    pl.kernel,
    out_type=jax.ShapeDtypeStruct((2, NN, CC), jnp.float32),
    mesh=_mesh,
    scratch_types=[
        pltpu.VMEM((max(CH0, CH1), CB), jnp.int32),
        pltpu.VMEM((max(CH0, CH1), CB), jnp.int32),
        pltpu.VMEM((CB, CC), jnp.float32),
        pltpu.VMEM((CB, CC), jnp.float32),
        pltpu.VMEM_SHARED((NPAD, CC), jnp.float32),
        pltpu.SemaphoreType.DMA,
        pltpu.SemaphoreType.DMA,
    ],
    compiler_params=pltpu.CompilerParams(use_tc_tiling_on_sc=False),
)
def _hop_kernel(g_hbm, src_hbm, dst_hbm, zeros_hbm, out_hbm,
                src_v, dst_v, rows_a, rows_b, acc, sem_a, sem_b):
    c = lax.axis_index("c")
    s = lax.axis_index("s")
    pltpu.sync_copy(zeros_hbm.at[pl.ds(s * ZR, ZR)], acc.at[pl.ds(s * ZR, ZR)])

    def stage(nch, base):
        pltpu.sync_copy(src_hbm.at[pl.ds(base, nch)], src_v.at[pl.ds(0, nch)])
        pltpu.sync_copy(dst_hbm.at[pl.ds(base, nch)], dst_v.at[pl.ds(0, nch)])

    @pl.when(c == 0)
    def _():
        stage(CH0, s * CH0)

    if CH1 > 0:
        @pl.when(c == 1)
        def _():
            stage(CH1, 16 * CH0 + s * CH1)

    plsc.subcore_barrier()

    def run_loop(nch):
        pltpu.async_copy(g_hbm.at[src_v.at[0]], rows_a, sem_a)

        def body(u, carry):
            j0 = 2 * u
            j1 = 2 * u + 1
            db = pltpu.async_copy(g_hbm.at[src_v.at[j1]], rows_b, sem_b)
            pltpu.make_async_copy(g_hbm.at[src_v.at[j0]], rows_a, sem_a).wait()

            @pl.when(u < nch // 2 - 1)
            def _():
                pltpu.async_copy(g_hbm.at[src_v.at[j0 + 2]], rows_a, sem_a)

            db.wait()
            return carry

        lax.fori_loop(0, nch // 2, body, 0)

    @pl.when(c == 0)
    def _():
        run_loop(CH0)

    if CH1 > 0:
        @pl.when(c == 1)
        def _():
            run_loop(CH1)

    plsc.subcore_barrier()

    @pl.when(s < 15)
    def _():
        pltpu.sync_copy(acc.at[pl.ds(s * CR, CR)],
                        out_hbm.at[c, pl.ds(s * CR, CR)])

    @pl.when(s == 15)
    def _():
        pltpu.sync_copy(acc.at[pl.ds(15 * CR, CRL)],
                        out_hbm.at[c, pl.ds(15 * CR, CRL)])


def _matmul_body(x_ref, w_ref, y_ref):
    y_ref[...] = jnp.dot(x_ref[...], w_ref[...],
                         preferred_element_type=jnp.float32)


_matmul = pl.pallas_call(
    _matmul_body,
    grid=(GRID,),
    in_specs=[
        pl.BlockSpec((BN, DD), lambda i: (i, 0)),
        pl.BlockSpec((DD, CC), lambda i: (0, 0)),
    ],
    out_specs=pl.BlockSpec((BN, CC), lambda i: (i, 0)),
    out_shape=jax.ShapeDtypeStruct((NN, CC), jnp.float32),
)


def _scale_body(y_ref, deg_ref, o_ref, nrm_ref):
    d = deg_ref[...]
    din = d[0, 0, :, 0] + d[1, 0, :, 0]
    dout = d[0, 1, :, 0] + d[1, 1, :, 0]
    nd = lax.rsqrt(jnp.maximum(din, 1.0))
    ns = lax.rsqrt(jnp.maximum(dout, 1.0))
    nrm_ref[...] = jnp.stack([nd, ns, ns * nd, ns], axis=-1)
    o_ref[...] = y_ref[...] * ns[:, None]


_scale = pl.pallas_call(
    _scale_body,
    grid=(GRID,),
    in_specs=[
        pl.BlockSpec((BN, CC), lambda i: (i, 0)),
        pl.BlockSpec((2, 2, BN, 16), lambda i: (0, 0, i, 0)),
    ],
    out_specs=[
        pl.BlockSpec((BN, CC), lambda i: (i, 0)),
        pl.BlockSpec((BN, 4), lambda i: (i, 0)),
    ],
    out_shape=[
        jax.ShapeDtypeStruct((NN, CC), jnp.float32),
        jax.ShapeDtypeStruct((NN, 4), jnp.float32),
    ],
)


def _mid_body(a_ref, nrm_ref, o_ref):
    s2 = nrm_ref[:, 2]
    a = a_ref[0] + a_ref[1]
    o_ref[...] = a * s2[:, None]


_mid = pl.pallas_call(
    _mid_body,
    grid=(GRID,),
    in_specs=[
        pl.BlockSpec((2, BN, CC), lambda i: (0, i, 0)),
        pl.BlockSpec((BN, 4), lambda i: (i, 0)),
    ],
    out_specs=pl.BlockSpec((BN, CC), lambda i: (i, 0)),
    out_shape=jax.ShapeDtypeStruct((NN, CC), jnp.float32),
)


def _fin_body(a_ref, nrm_ref, p_ref, l_ref, e_ref):
    nd = nrm_ref[:, 0]
    logits = (a_ref[0] + a_ref[1]) * nd[:, None]
    m = jnp.max(logits, axis=1, keepdims=True)
    ex = jnp.exp(logits - m)
    p = ex / jnp.sum(ex, axis=1, keepdims=True)
    l_ref[...] = logits
    p_ref[...] = p
    ent_sum = -jnp.sum(p * jnp.log(p + 1e-12))
    i = pl.program_id(0)

    @pl.when(i == 0)
    def _():
        e_ref[...] = jnp.reshape(ent_sum, (1, 1))

    @pl.when(i > 0)
    def _():
        e_ref[...] = e_ref[...] + ent_sum

    @pl.when(i == GRID - 1)
    def _():
        e_ref[...] = e_ref[...] * (1.0 / NN)


_fin = pl.pallas_call(
    _fin_body,
    grid=(GRID,),
    in_specs=[
        pl.BlockSpec((2, BN, CC), lambda i: (0, i, 0)),
        pl.BlockSpec((BN, 4), lambda i: (i, 0)),
    ],
    out_specs=[
        pl.BlockSpec((BN, CC), lambda i: (i, 0)),
        pl.BlockSpec((BN, CC), lambda i: (i, 0)),
        pl.BlockSpec((1, 1), lambda i: (0, 0)),
    ],
    out_shape=[
        jax.ShapeDtypeStruct((NN, CC), jnp.float32),
        jax.ShapeDtypeStruct((NN, CC), jnp.float32),
        jax.ShapeDtypeStruct((1, 1), jnp.float32),
    ],
)


def kernel(features, edge_index, W):
    sink_pad = jnp.full((EPAD - EE,), SINK, jnp.int32)
    zero_pad = jnp.zeros((EPAD - EE,), jnp.int32)
    src_deg = jnp.concatenate([edge_index[0], sink_pad]).reshape(ROWS, CB)
    src_hop = jnp.concatenate([edge_index[0], zero_pad]).reshape(ROWS, CB)
    dst_p = jnp.concatenate([edge_index[1], sink_pad]).reshape(ROWS, CB)
    ones16 = jnp.ones((CB, 16), jnp.float32)
    zeros16 = jnp.zeros((NPAD, 16), jnp.float32)
    zeros64 = jnp.zeros((NPAD, CC), jnp.float32)

    degp = _deg_kernel(src_deg, dst_p, ones16, zeros16)
    y0 = _matmul(features, W)
    g1, nrm = _scale(y0, degp)
    a1 = _hop_kernel(g1, src_hop, dst_p, zeros64)
    g2 = _mid(a1, nrm)
    a2 = _hop_kernel(g2, src_hop, dst_p, zeros64)
    probs, logits, ent = _fin(a2, nrm)
    return probs, logits, ent[0, 0]


# speedup vs baseline: 1.5127x; 1.5127x over previous
import functools

import jax
import jax.numpy as jnp
from jax import lax
from jax.experimental import pallas as pl
from jax.experimental.pallas import tpu as pltpu
from jax.experimental.pallas import tpu_sc as plsc

NN = 10000
EE = 160000
DD = 256
CC = 64
SINK = NN
NPAD = 10112
ZR = NPAD // 16
CR = 632
CRL = NN - 15 * CR
NW = 32
CB = 128
CHUNKS = 40
CH0 = 56
CH1 = 24
EPAD = NW * CHUNKS * CB
ROWS = EPAD // CB
BN = 1000
GRID = NN // BN

_mesh = plsc.VectorSubcoreMesh(core_axis_name="c", subcore_axis_name="s")


@functools.partial(
    pl.kernel,
    out_type=jax.ShapeDtypeStruct((2, 2, NN, 16), jnp.float32),
    mesh=_mesh,
    scratch_types=[
        pltpu.VMEM((CHUNKS, CB), jnp.int32),
        pltpu.VMEM((CHUNKS, CB), jnp.int32),
        pltpu.VMEM((CB, 16), jnp.float32),
        pltpu.VMEM_SHARED((NPAD, 16), jnp.float32),
        pltpu.VMEM_SHARED((NPAD, 16), jnp.float32),
    ],
    compiler_params=pltpu.CompilerParams(use_tc_tiling_on_sc=False),
)
def _deg_kernel(src_hbm, dst_hbm, ones_hbm, zeros_hbm, out_hbm,
                src_v, dst_v, ones_v, acc_in, acc_out):
    c = lax.axis_index("c")
    s = lax.axis_index("s")
    wid = c * 16 + s
    pltpu.sync_copy(ones_hbm, ones_v)
    pltpu.sync_copy(zeros_hbm.at[pl.ds(s * ZR, ZR)], acc_in.at[pl.ds(s * ZR, ZR)])
    pltpu.sync_copy(zeros_hbm.at[pl.ds(s * ZR, ZR)], acc_out.at[pl.ds(s * ZR, ZR)])
    pltpu.sync_copy(src_hbm.at[pl.ds(wid * CHUNKS, CHUNKS)], src_v)
    pltpu.sync_copy(dst_hbm.at[pl.ds(wid * CHUNKS, CHUNKS)], dst_v)
    plsc.subcore_barrier()

    def body(j, carry):
        pltpu.sync_copy(ones_v, acc_in.at[dst_v.at[j]], add=True)
        pltpu.sync_copy(ones_v, acc_out.at[src_v.at[j]], add=True)
        return carry

    lax.fori_loop(0, CHUNKS, body, 0)
    plsc.subcore_barrier()

    @pl.when(s < 15)
    def _():
        pltpu.sync_copy(acc_in.at[pl.ds(s * CR, CR)],
                        out_hbm.at[c, 0, pl.ds(s * CR, CR)])
        pltpu.sync_copy(acc_out.at[pl.ds(s * CR, CR)],
                        out_hbm.at[c, 1, pl.ds(s * CR, CR)])

    @pl.when(s == 15)
    def _():
        pltpu.sync_copy(acc_in.at[pl.ds(15 * CR, CRL)],
                        out_hbm.at[c, 0, pl.ds(15 * CR, CRL)])
        pltpu.sync_copy(acc_out.at[pl.ds(15 * CR, CRL)],
                        out_hbm.at[c, 1, pl.ds(15 * CR, CRL)])


@functools.partial(
    pl.kernel,
    out_type=jax.ShapeDtypeStruct((2, NN, CC), jnp.float32),
    mesh=_mesh,
    scratch_types=[
        pltpu.VMEM((max(CH0, CH1), CB), jnp.int32),
        pltpu.VMEM((max(CH0, CH1), CB), jnp.int32),
        pltpu.VMEM((CB, CC), jnp.float32),
        pltpu.VMEM((CB, CC), jnp.float32),
        pltpu.VMEM_SHARED((NPAD, CC), jnp.float32),
        pltpu.VMEM_SHARED((NN, CC), jnp.float32),
        pltpu.SemaphoreType.DMA,
        pltpu.SemaphoreType.DMA,
    ],
    compiler_params=pltpu.CompilerParams(use_tc_tiling_on_sc=False),
)
def _hop_kernel(g_hbm, src_hbm, dst_hbm, zeros_hbm, out_hbm,
                src_v, dst_v, rows_a, rows_b, acc, g_s, sem_a, sem_b):
    c = lax.axis_index("c")
    s = lax.axis_index("s")
    pltpu.sync_copy(zeros_hbm.at[pl.ds(s * ZR, ZR)], acc.at[pl.ds(s * ZR, ZR)])

    @pl.when(s < 15)
    def _():
        pltpu.sync_copy(g_hbm.at[pl.ds(s * CR, CR)], g_s.at[pl.ds(s * CR, CR)])

    @pl.when(s == 15)
    def _():
        pltpu.sync_copy(g_hbm.at[pl.ds(15 * CR, CRL)], g_s.at[pl.ds(15 * CR, CRL)])

    def stage(nch, base):
        pltpu.sync_copy(src_hbm.at[pl.ds(base, nch)], src_v.at[pl.ds(0, nch)])
        pltpu.sync_copy(dst_hbm.at[pl.ds(base, nch)], dst_v.at[pl.ds(0, nch)])

    @pl.when(c == 0)
    def _():
        stage(CH0, s * CH0)

    if CH1 > 0:
        @pl.when(c == 1)
        def _():
            stage(CH1, 16 * CH0 + s * CH1)

    plsc.subcore_barrier()

    def run_loop(nch):
        pltpu.async_copy(g_s.at[src_v.at[0]], rows_a, sem_a)

        def body(u, carry):
            j0 = 2 * u
            j1 = 2 * u + 1
            db = pltpu.async_copy(g_s.at[src_v.at[j1]], rows_b, sem_b)
            pltpu.make_async_copy(g_s.at[src_v.at[j0]], rows_a, sem_a).wait()
            pltpu.sync_copy(rows_a, acc.at[dst_v.at[j0]], add=True)

            @pl.when(u < nch // 2 - 1)
            def _():
                pltpu.async_copy(g_s.at[src_v.at[j0 + 2]], rows_a, sem_a)

            db.wait()
            pltpu.sync_copy(rows_b, acc.at[dst_v.at[j1]], add=True)
            return carry

        lax.fori_loop(0, nch // 2, body, 0)

    @pl.when(c == 0)
    def _():
        run_loop(CH0)

    if CH1 > 0:
        @pl.when(c == 1)
        def _():
            run_loop(CH1)

    plsc.subcore_barrier()

    @pl.when(s < 15)
    def _():
        pltpu.sync_copy(acc.at[pl.ds(s * CR, CR)],
                        out_hbm.at[c, pl.ds(s * CR, CR)])

    @pl.when(s == 15)
    def _():
        pltpu.sync_copy(acc.at[pl.ds(15 * CR, CRL)],
                        out_hbm.at[c, pl.ds(15 * CR, CRL)])


def _matmul_body(x_ref, w_ref, y_ref):
    y_ref[...] = jnp.dot(x_ref[...], w_ref[...],
                         preferred_element_type=jnp.float32)


_matmul = pl.pallas_call(
    _matmul_body,
    grid=(GRID,),
    in_specs=[
        pl.BlockSpec((BN, DD), lambda i: (i, 0)),
        pl.BlockSpec((DD, CC), lambda i: (0, 0)),
    ],
    out_specs=pl.BlockSpec((BN, CC), lambda i: (i, 0)),
    out_shape=jax.ShapeDtypeStruct((NN, CC), jnp.float32),
)


def _scale_body(y_ref, deg_ref, o_ref, nrm_ref):
    d = deg_ref[...]
    din = d[0, 0, :, 0] + d[1, 0, :, 0]
    dout = d[0, 1, :, 0] + d[1, 1, :, 0]
    nd = lax.rsqrt(jnp.maximum(din, 1.0))
    ns = lax.rsqrt(jnp.maximum(dout, 1.0))
    nrm_ref[...] = jnp.stack([nd, ns, ns * nd, ns], axis=-1)
    o_ref[...] = y_ref[...] * ns[:, None]


_scale = pl.pallas_call(
    _scale_body,
    grid=(GRID,),
    in_specs=[
        pl.BlockSpec((BN, CC), lambda i: (i, 0)),
        pl.BlockSpec((2, 2, BN, 16), lambda i: (0, 0, i, 0)),
    ],
    out_specs=[
        pl.BlockSpec((BN, CC), lambda i: (i, 0)),
        pl.BlockSpec((BN, 4), lambda i: (i, 0)),
    ],
    out_shape=[
        jax.ShapeDtypeStruct((NN, CC), jnp.float32),
        jax.ShapeDtypeStruct((NN, 4), jnp.float32),
    ],
)


def _mid_body(a_ref, nrm_ref, o_ref):
    s2 = nrm_ref[:, 2]
    a = a_ref[0] + a_ref[1]
    o_ref[...] = a * s2[:, None]


_mid = pl.pallas_call(
    _mid_body,
    grid=(GRID,),
    in_specs=[
        pl.BlockSpec((2, BN, CC), lambda i: (0, i, 0)),
        pl.BlockSpec((BN, 4), lambda i: (i, 0)),
    ],
    out_specs=pl.BlockSpec((BN, CC), lambda i: (i, 0)),
    out_shape=jax.ShapeDtypeStruct((NN, CC), jnp.float32),
)


def _fin_body(a_ref, nrm_ref, p_ref, l_ref, e_ref):
    nd = nrm_ref[:, 0]
    logits = (a_ref[0] + a_ref[1]) * nd[:, None]
    m = jnp.max(logits, axis=1, keepdims=True)
    ex = jnp.exp(logits - m)
    p = ex / jnp.sum(ex, axis=1, keepdims=True)
    l_ref[...] = logits
    p_ref[...] = p
    ent_sum = -jnp.sum(p * jnp.log(p + 1e-12))
    i = pl.program_id(0)

    @pl.when(i == 0)
    def _():
        e_ref[...] = jnp.reshape(ent_sum, (1, 1))

    @pl.when(i > 0)
    def _():
        e_ref[...] = e_ref[...] + ent_sum

    @pl.when(i == GRID - 1)
    def _():
        e_ref[...] = e_ref[...] * (1.0 / NN)


_fin = pl.pallas_call(
    _fin_body,
    grid=(GRID,),
    in_specs=[
        pl.BlockSpec((2, BN, CC), lambda i: (0, i, 0)),
        pl.BlockSpec((BN, 4), lambda i: (i, 0)),
    ],
    out_specs=[
        pl.BlockSpec((BN, CC), lambda i: (i, 0)),
        pl.BlockSpec((BN, CC), lambda i: (i, 0)),
        pl.BlockSpec((1, 1), lambda i: (0, 0)),
    ],
    out_shape=[
        jax.ShapeDtypeStruct((NN, CC), jnp.float32),
        jax.ShapeDtypeStruct((NN, CC), jnp.float32),
        jax.ShapeDtypeStruct((1, 1), jnp.float32),
    ],
)


def kernel(features, edge_index, W):
    sink_pad = jnp.full((EPAD - EE,), SINK, jnp.int32)
    zero_pad = jnp.zeros((EPAD - EE,), jnp.int32)
    src_deg = jnp.concatenate([edge_index[0], sink_pad]).reshape(ROWS, CB)
    src_hop = jnp.concatenate([edge_index[0], zero_pad]).reshape(ROWS, CB)
    dst_p = jnp.concatenate([edge_index[1], sink_pad]).reshape(ROWS, CB)
    ones16 = jnp.ones((CB, 16), jnp.float32)
    zeros16 = jnp.zeros((NPAD, 16), jnp.float32)
    zeros64 = jnp.zeros((NPAD, CC), jnp.float32)

    degp = _deg_kernel(src_deg, dst_p, ones16, zeros16)
    y0 = _matmul(features, W)
    g1, nrm = _scale(y0, degp)
    a1 = _hop_kernel(g1, src_hop, dst_p, zeros64)
    g2 = _mid(a1, nrm)
    a2 = _hop_kernel(g2, src_hop, dst_p, zeros64)
    probs, logits, ent = _fin(a2, nrm)
    return probs, logits, ent[0, 0]


# speedup vs baseline: 1.6722x; 1.1055x over previous
import functools

import jax
import jax.numpy as jnp
from jax import lax
from jax.experimental import pallas as pl
from jax.experimental.pallas import tpu as pltpu
from jax.experimental.pallas import tpu_sc as plsc

NN = 10000
EE = 160000
DD = 256
CC = 64
SINK = NN
NPAD = 10112
ZR = NPAD // 16
CR = 632
CRL = NN - 15 * CR
NW = 32
CB = 128
CHUNKS = 40
CH0 = 40
CH1 = 40
EPAD = NW * CHUNKS * CB
ROWS = EPAD // CB
BN = 1000
GRID = NN // BN

_mesh = plsc.VectorSubcoreMesh(core_axis_name="c", subcore_axis_name="s")


@functools.partial(
    pl.kernel,
    out_type=jax.ShapeDtypeStruct((2, 2, NN, 16), jnp.float32),
    mesh=_mesh,
    scratch_types=[
        pltpu.VMEM((CHUNKS, CB), jnp.int32),
        pltpu.VMEM((CHUNKS, CB), jnp.int32),
        pltpu.VMEM((CB, 16), jnp.float32),
        pltpu.VMEM_SHARED((NPAD, 16), jnp.float32),
        pltpu.VMEM_SHARED((NPAD, 16), jnp.float32),
    ],
    compiler_params=pltpu.CompilerParams(use_tc_tiling_on_sc=False),
)
def _deg_kernel(src_hbm, dst_hbm, ones_hbm, zeros_hbm, out_hbm,
                src_v, dst_v, ones_v, acc_in, acc_out):
    c = lax.axis_index("c")
    s = lax.axis_index("s")
    wid = c * 16 + s
    pltpu.sync_copy(ones_hbm, ones_v)
    pltpu.sync_copy(zeros_hbm.at[pl.ds(s * ZR, ZR)], acc_in.at[pl.ds(s * ZR, ZR)])
    pltpu.sync_copy(zeros_hbm.at[pl.ds(s * ZR, ZR)], acc_out.at[pl.ds(s * ZR, ZR)])
    pltpu.sync_copy(src_hbm.at[pl.ds(wid * CHUNKS, CHUNKS)], src_v)
    pltpu.sync_copy(dst_hbm.at[pl.ds(wid * CHUNKS, CHUNKS)], dst_v)
    plsc.subcore_barrier()

    def body(j, carry):
        pltpu.sync_copy(ones_v, acc_in.at[dst_v.at[j]], add=True)
        pltpu.sync_copy(ones_v, acc_out.at[src_v.at[j]], add=True)
        return carry

    lax.fori_loop(0, CHUNKS, body, 0)
    plsc.subcore_barrier()

    @pl.when(s < 15)
    def _():
        pltpu.sync_copy(acc_in.at[pl.ds(s * CR, CR)],
                        out_hbm.at[c, 0, pl.ds(s * CR, CR)])
        pltpu.sync_copy(acc_out.at[pl.ds(s * CR, CR)],
                        out_hbm.at[c, 1, pl.ds(s * CR, CR)])

    @pl.when(s == 15)
    def _():
        pltpu.sync_copy(acc_in.at[pl.ds(15 * CR, CRL)],
                        out_hbm.at[c, 0, pl.ds(15 * CR, CRL)])
        pltpu.sync_copy(acc_out.at[pl.ds(15 * CR, CRL)],
                        out_hbm.at[c, 1, pl.ds(15 * CR, CRL)])


@functools.partial(
    pl.kernel,
    out_type=jax.ShapeDtypeStruct((2, NN, CC), jnp.float32),
    mesh=_mesh,
    scratch_types=[
        pltpu.VMEM((max(CH0, CH1), CB), jnp.int32),
        pltpu.VMEM((max(CH0, CH1), CB), jnp.int32),
        pltpu.VMEM((CB, CC), jnp.float32),
        pltpu.VMEM((CB, CC), jnp.float32),
        pltpu.VMEM_SHARED((NPAD, CC), jnp.float32),
        pltpu.VMEM_SHARED((NN, CC), jnp.float32),
        pltpu.SemaphoreType.DMA,
        pltpu.SemaphoreType.DMA,
    ],
    compiler_params=pltpu.CompilerParams(use_tc_tiling_on_sc=False),
)
def _hop_kernel(g_hbm, src_hbm, dst_hbm, zeros_hbm, out_hbm,
                src_v, dst_v, rows_a, rows_b, acc, g_s, sem_a, sem_b):
    c = lax.axis_index("c")
    s = lax.axis_index("s")
    pltpu.sync_copy(zeros_hbm.at[pl.ds(s * ZR, ZR)], acc.at[pl.ds(s * ZR, ZR)])

    @pl.when(s < 15)
    def _():
        pltpu.sync_copy(g_hbm.at[pl.ds(s * CR, CR)], g_s.at[pl.ds(s * CR, CR)])

    @pl.when(s == 15)
    def _():
        pltpu.sync_copy(g_hbm.at[pl.ds(15 * CR, CRL)], g_s.at[pl.ds(15 * CR, CRL)])

    def stage(nch, base):
        pltpu.sync_copy(src_hbm.at[pl.ds(base, nch)], src_v.at[pl.ds(0, nch)])
        pltpu.sync_copy(dst_hbm.at[pl.ds(base, nch)], dst_v.at[pl.ds(0, nch)])

    @pl.when(c == 0)
    def _():
        stage(CH0, s * CH0)

    if CH1 > 0:
        @pl.when(c == 1)
        def _():
            stage(CH1, 16 * CH0 + s * CH1)

    plsc.subcore_barrier()

    def run_loop(nch):
        pltpu.async_copy(g_s.at[src_v.at[0]], rows_a, sem_a)

        def body(u, carry):
            j0 = 2 * u
            j1 = 2 * u + 1
            db = pltpu.async_copy(g_s.at[src_v.at[j1]], rows_b, sem_b)
            pltpu.make_async_copy(g_s.at[src_v.at[j0]], rows_a, sem_a).wait()
            pltpu.sync_copy(rows_a, acc.at[dst_v.at[j0]], add=True)

            @pl.when(u < nch // 2 - 1)
            def _():
                pltpu.async_copy(g_s.at[src_v.at[j0 + 2]], rows_a, sem_a)

            db.wait()
            pltpu.sync_copy(rows_b, acc.at[dst_v.at[j1]], add=True)
            return carry

        lax.fori_loop(0, nch // 2, body, 0)

    @pl.when(c == 0)
    def _():
        run_loop(CH0)

    if CH1 > 0:
        @pl.when(c == 1)
        def _():
            run_loop(CH1)

    plsc.subcore_barrier()

    @pl.when(s < 15)
    def _():
        pltpu.sync_copy(acc.at[pl.ds(s * CR, CR)],
                        out_hbm.at[c, pl.ds(s * CR, CR)])

    @pl.when(s == 15)
    def _():
        pltpu.sync_copy(acc.at[pl.ds(15 * CR, CRL)],
                        out_hbm.at[c, pl.ds(15 * CR, CRL)])


def _matmul_body(x_ref, w_ref, y_ref):
    y_ref[...] = jnp.dot(x_ref[...], w_ref[...],
                         preferred_element_type=jnp.float32)


_matmul = pl.pallas_call(
    _matmul_body,
    grid=(GRID,),
    in_specs=[
        pl.BlockSpec((BN, DD), lambda i: (i, 0)),
        pl.BlockSpec((DD, CC), lambda i: (0, 0)),
    ],
    out_specs=pl.BlockSpec((BN, CC), lambda i: (i, 0)),
    out_shape=jax.ShapeDtypeStruct((NN, CC), jnp.float32),
)


def _scale_body(y_ref, deg_ref, o_ref, nrm_ref):
    d = deg_ref[...]
    din = d[0, 0, :, 0] + d[1, 0, :, 0]
    dout = d[0, 1, :, 0] + d[1, 1, :, 0]
    nd = lax.rsqrt(jnp.maximum(din, 1.0))
    ns = lax.rsqrt(jnp.maximum(dout, 1.0))
    nrm_ref[...] = jnp.stack([nd, ns, ns * nd, ns], axis=-1)
    o_ref[...] = y_ref[...] * ns[:, None]


_scale = pl.pallas_call(
    _scale_body,
    grid=(GRID,),
    in_specs=[
        pl.BlockSpec((BN, CC), lambda i: (i, 0)),
        pl.BlockSpec((2, 2, BN, 16), lambda i: (0, 0, i, 0)),
    ],
    out_specs=[
        pl.BlockSpec((BN, CC), lambda i: (i, 0)),
        pl.BlockSpec((BN, 4), lambda i: (i, 0)),
    ],
    out_shape=[
        jax.ShapeDtypeStruct((NN, CC), jnp.float32),
        jax.ShapeDtypeStruct((NN, 4), jnp.float32),
    ],
)


def _mid_body(a_ref, nrm_ref, o_ref):
    s2 = nrm_ref[:, 2]
    a = a_ref[0] + a_ref[1]
    o_ref[...] = a * s2[:, None]


_mid = pl.pallas_call(
    _mid_body,
    grid=(GRID,),
    in_specs=[
        pl.BlockSpec((2, BN, CC), lambda i: (0, i, 0)),
        pl.BlockSpec((BN, 4), lambda i: (i, 0)),
    ],
    out_specs=pl.BlockSpec((BN, CC), lambda i: (i, 0)),
    out_shape=jax.ShapeDtypeStruct((NN, CC), jnp.float32),
)


def _fin_body(a_ref, nrm_ref, p_ref, l_ref, e_ref):
    nd = nrm_ref[:, 0]
    logits = (a_ref[0] + a_ref[1]) * nd[:, None]
    m = jnp.max(logits, axis=1, keepdims=True)
    ex = jnp.exp(logits - m)
    p = ex / jnp.sum(ex, axis=1, keepdims=True)
    l_ref[...] = logits
    p_ref[...] = p
    ent_sum = -jnp.sum(p * jnp.log(p + 1e-12))
    i = pl.program_id(0)

    @pl.when(i == 0)
    def _():
        e_ref[...] = jnp.reshape(ent_sum, (1, 1))

    @pl.when(i > 0)
    def _():
        e_ref[...] = e_ref[...] + ent_sum

    @pl.when(i == GRID - 1)
    def _():
        e_ref[...] = e_ref[...] * (1.0 / NN)


_fin = pl.pallas_call(
    _fin_body,
    grid=(GRID,),
    in_specs=[
        pl.BlockSpec((2, BN, CC), lambda i: (0, i, 0)),
        pl.BlockSpec((BN, 4), lambda i: (i, 0)),
    ],
    out_specs=[
        pl.BlockSpec((BN, CC), lambda i: (i, 0)),
        pl.BlockSpec((BN, CC), lambda i: (i, 0)),
        pl.BlockSpec((1, 1), lambda i: (0, 0)),
    ],
    out_shape=[
        jax.ShapeDtypeStruct((NN, CC), jnp.float32),
        jax.ShapeDtypeStruct((NN, CC), jnp.float32),
        jax.ShapeDtypeStruct((1, 1), jnp.float32),
    ],
)


def kernel(features, edge_index, W):
    sink_pad = jnp.full((EPAD - EE,), SINK, jnp.int32)
    zero_pad = jnp.zeros((EPAD - EE,), jnp.int32)
    src_deg = jnp.concatenate([edge_index[0], sink_pad]).reshape(ROWS, CB)
    src_hop = jnp.concatenate([edge_index[0], zero_pad]).reshape(ROWS, CB)
    dst_p = jnp.concatenate([edge_index[1], sink_pad]).reshape(ROWS, CB)
    ones16 = jnp.ones((CB, 16), jnp.float32)
    zeros16 = jnp.zeros((NPAD, 16), jnp.float32)
    zeros64 = jnp.zeros((NPAD, CC), jnp.float32)

    degp = _deg_kernel(src_deg, dst_p, ones16, zeros16)
    y0 = _matmul(features, W)
    g1, nrm = _scale(y0, degp)
    a1 = _hop_kernel(g1, src_hop, dst_p, zeros64)
    g2 = _mid(a1, nrm)
    a2 = _hop_kernel(g2, src_hop, dst_p, zeros64)
    probs, logits, ent = _fin(a2, nrm)
    return probs, logits, ent[0, 0]


# speedup vs baseline: 2.0066x; 1.2000x over previous
import functools

import jax
import jax.numpy as jnp
from jax import lax
from jax.experimental import pallas as pl
from jax.experimental.pallas import tpu as pltpu
from jax.experimental.pallas import tpu_sc as plsc

NN = 10000
EE = 160000
DD = 256
CC = 64
SINK = NN
NPAD = 10112
ZR = NPAD // 16
CR = 632
CRL = NN - 15 * CR
NW = 32
CB = 128
CHUNKS = 40
CH0 = 40
CH1 = 40
EPAD = NW * CHUNKS * CB
ROWS = EPAD // CB
BN = 1000
GRID = NN // BN

_mesh = plsc.VectorSubcoreMesh(core_axis_name="c", subcore_axis_name="s")


@functools.partial(
    pl.kernel,
    out_type=jax.ShapeDtypeStruct((2, 2, NN, 16), jnp.float32),
    mesh=_mesh,
    scratch_types=[
        pltpu.VMEM((CHUNKS, CB), jnp.int32),
        pltpu.VMEM((CHUNKS, CB), jnp.int32),
        pltpu.VMEM((CB, 16), jnp.float32),
        pltpu.VMEM_SHARED((NPAD, 16), jnp.float32),
        pltpu.VMEM_SHARED((NPAD, 16), jnp.float32),
    ],
    compiler_params=pltpu.CompilerParams(use_tc_tiling_on_sc=False),
)
def _deg_kernel(src_hbm, dst_hbm, ones_hbm, zeros_hbm, out_hbm,
                src_v, dst_v, ones_v, acc_in, acc_out):
    c = lax.axis_index("c")
    s = lax.axis_index("s")
    wid = c * 16 + s
    pltpu.sync_copy(ones_hbm, ones_v)
    pltpu.sync_copy(zeros_hbm.at[pl.ds(s * ZR, ZR)], acc_in.at[pl.ds(s * ZR, ZR)])
    pltpu.sync_copy(zeros_hbm.at[pl.ds(s * ZR, ZR)], acc_out.at[pl.ds(s * ZR, ZR)])
    pltpu.sync_copy(src_hbm.at[pl.ds(wid * CHUNKS, CHUNKS)], src_v)
    pltpu.sync_copy(dst_hbm.at[pl.ds(wid * CHUNKS, CHUNKS)], dst_v)
    plsc.subcore_barrier()

    def body(j, carry):
        pltpu.sync_copy(ones_v, acc_in.at[dst_v.at[j]], add=True)
        pltpu.sync_copy(ones_v, acc_out.at[src_v.at[j]], add=True)
        return carry

    lax.fori_loop(0, CHUNKS, body, 0)
    plsc.subcore_barrier()

    @pl.when(s < 15)
    def _():
        pltpu.sync_copy(acc_in.at[pl.ds(s * CR, CR)],
                        out_hbm.at[c, 0, pl.ds(s * CR, CR)])
        pltpu.sync_copy(acc_out.at[pl.ds(s * CR, CR)],
                        out_hbm.at[c, 1, pl.ds(s * CR, CR)])

    @pl.when(s == 15)
    def _():
        pltpu.sync_copy(acc_in.at[pl.ds(15 * CR, CRL)],
                        out_hbm.at[c, 0, pl.ds(15 * CR, CRL)])
        pltpu.sync_copy(acc_out.at[pl.ds(15 * CR, CRL)],
                        out_hbm.at[c, 1, pl.ds(15 * CR, CRL)])


@functools.partial(
    pl.kernel,
    out_type=jax.ShapeDtypeStruct((2, NN, CC), jnp.bfloat16),
    mesh=_mesh,
    scratch_types=[
        pltpu.VMEM((max(CH0, CH1), CB), jnp.int32),
        pltpu.VMEM((max(CH0, CH1), CB), jnp.int32),
        pltpu.VMEM((CB, CC), jnp.bfloat16),
        pltpu.VMEM((CB, CC), jnp.bfloat16),
        pltpu.VMEM_SHARED((NPAD, CC), jnp.bfloat16),
        pltpu.VMEM_SHARED((NN, CC), jnp.bfloat16),
        pltpu.SemaphoreType.DMA,
        pltpu.SemaphoreType.DMA,
    ],
    compiler_params=pltpu.CompilerParams(use_tc_tiling_on_sc=False),
)
def _hop_kernel(g_hbm, src_hbm, dst_hbm, zeros_hbm, out_hbm,
                src_v, dst_v, rows_a, rows_b, acc, g_s, sem_a, sem_b):
    c = lax.axis_index("c")
    s = lax.axis_index("s")
    pltpu.sync_copy(zeros_hbm.at[pl.ds(s * ZR, ZR)], acc.at[pl.ds(s * ZR, ZR)])

    @pl.when(s < 15)
    def _():
        pltpu.sync_copy(g_hbm.at[pl.ds(s * CR, CR)], g_s.at[pl.ds(s * CR, CR)])

    @pl.when(s == 15)
    def _():
        pltpu.sync_copy(g_hbm.at[pl.ds(15 * CR, CRL)], g_s.at[pl.ds(15 * CR, CRL)])

    def stage(nch, base):
        pltpu.sync_copy(src_hbm.at[pl.ds(base, nch)], src_v.at[pl.ds(0, nch)])
        pltpu.sync_copy(dst_hbm.at[pl.ds(base, nch)], dst_v.at[pl.ds(0, nch)])

    @pl.when(c == 0)
    def _():
        stage(CH0, s * CH0)

    if CH1 > 0:
        @pl.when(c == 1)
        def _():
            stage(CH1, 16 * CH0 + s * CH1)

    plsc.subcore_barrier()

    def run_loop(nch):
        pltpu.async_copy(g_s.at[src_v.at[0]], rows_a, sem_a)

        def body(u, carry):
            j0 = 2 * u
            j1 = 2 * u + 1
            db = pltpu.async_copy(g_s.at[src_v.at[j1]], rows_b, sem_b)
            pltpu.make_async_copy(g_s.at[src_v.at[j0]], rows_a, sem_a).wait()
            pltpu.sync_copy(rows_a, acc.at[dst_v.at[j0]], add=True)

            @pl.when(u < nch // 2 - 1)
            def _():
                pltpu.async_copy(g_s.at[src_v.at[j0 + 2]], rows_a, sem_a)

            db.wait()
            pltpu.sync_copy(rows_b, acc.at[dst_v.at[j1]], add=True)
            return carry

        lax.fori_loop(0, nch // 2, body, 0)

    @pl.when(c == 0)
    def _():
        run_loop(CH0)

    if CH1 > 0:
        @pl.when(c == 1)
        def _():
            run_loop(CH1)

    plsc.subcore_barrier()

    @pl.when(s < 15)
    def _():
        pltpu.sync_copy(acc.at[pl.ds(s * CR, CR)],
                        out_hbm.at[c, pl.ds(s * CR, CR)])

    @pl.when(s == 15)
    def _():
        pltpu.sync_copy(acc.at[pl.ds(15 * CR, CRL)],
                        out_hbm.at[c, pl.ds(15 * CR, CRL)])


def _matmul_body(x_ref, w_ref, y_ref):
    y_ref[...] = jnp.dot(x_ref[...], w_ref[...],
                         preferred_element_type=jnp.float32)


_matmul = pl.pallas_call(
    _matmul_body,
    grid=(GRID,),
    in_specs=[
        pl.BlockSpec((BN, DD), lambda i: (i, 0)),
        pl.BlockSpec((DD, CC), lambda i: (0, 0)),
    ],
    out_specs=pl.BlockSpec((BN, CC), lambda i: (i, 0)),
    out_shape=jax.ShapeDtypeStruct((NN, CC), jnp.float32),
)


def _scale_body(y_ref, deg_ref, o_ref, nrm_ref):
    d = deg_ref[...]
    din = d[0, 0, :, 0] + d[1, 0, :, 0]
    dout = d[0, 1, :, 0] + d[1, 1, :, 0]
    nd = lax.rsqrt(jnp.maximum(din, 1.0))
    ns = lax.rsqrt(jnp.maximum(dout, 1.0))
    nrm_ref[...] = jnp.stack([nd, ns, ns * nd, ns], axis=-1)
    o_ref[...] = (y_ref[...] * ns[:, None]).astype(jnp.bfloat16)


_scale = pl.pallas_call(
    _scale_body,
    grid=(GRID,),
    in_specs=[
        pl.BlockSpec((BN, CC), lambda i: (i, 0)),
        pl.BlockSpec((2, 2, BN, 16), lambda i: (0, 0, i, 0)),
    ],
    out_specs=[
        pl.BlockSpec((BN, CC), lambda i: (i, 0)),
        pl.BlockSpec((BN, 4), lambda i: (i, 0)),
    ],
    out_shape=[
        jax.ShapeDtypeStruct((NN, CC), jnp.bfloat16),
        jax.ShapeDtypeStruct((NN, 4), jnp.float32),
    ],
)


def _mid_body(a_ref, nrm_ref, o_ref):
    s2 = nrm_ref[:, 2]
    a = a_ref[0].astype(jnp.float32) + a_ref[1].astype(jnp.float32)
    o_ref[...] = (a * s2[:, None]).astype(jnp.bfloat16)


_mid = pl.pallas_call(
    _mid_body,
    grid=(GRID,),
    in_specs=[
        pl.BlockSpec((2, BN, CC), lambda i: (0, i, 0)),
        pl.BlockSpec((BN, 4), lambda i: (i, 0)),
    ],
    out_specs=pl.BlockSpec((BN, CC), lambda i: (i, 0)),
    out_shape=jax.ShapeDtypeStruct((NN, CC), jnp.bfloat16),
)


def _fin_body(a_ref, nrm_ref, p_ref, l_ref, e_ref):
    nd = nrm_ref[:, 0]
    a = a_ref[0].astype(jnp.float32) + a_ref[1].astype(jnp.float32)
    logits = a * nd[:, None]
    m = jnp.max(logits, axis=1, keepdims=True)
    ex = jnp.exp(logits - m)
    p = ex / jnp.sum(ex, axis=1, keepdims=True)
    l_ref[...] = logits
    p_ref[...] = p
    ent_sum = -jnp.sum(p * jnp.log(p + 1e-12))
    i = pl.program_id(0)

    @pl.when(i == 0)
    def _():
        e_ref[...] = jnp.reshape(ent_sum, (1, 1))

    @pl.when(i > 0)
    def _():
        e_ref[...] = e_ref[...] + ent_sum

    @pl.when(i == GRID - 1)
    def _():
        e_ref[...] = e_ref[...] * (1.0 / NN)


_fin = pl.pallas_call(
    _fin_body,
    grid=(GRID,),
    in_specs=[
        pl.BlockSpec((2, BN, CC), lambda i: (0, i, 0)),
        pl.BlockSpec((BN, 4), lambda i: (i, 0)),
    ],
    out_specs=[
        pl.BlockSpec((BN, CC), lambda i: (i, 0)),
        pl.BlockSpec((BN, CC), lambda i: (i, 0)),
        pl.BlockSpec((1, 1), lambda i: (0, 0)),
    ],
    out_shape=[
        jax.ShapeDtypeStruct((NN, CC), jnp.float32),
        jax.ShapeDtypeStruct((NN, CC), jnp.float32),
        jax.ShapeDtypeStruct((1, 1), jnp.float32),
    ],
)


def kernel(features, edge_index, W):
    sink_pad = jnp.full((EPAD - EE,), SINK, jnp.int32)
    zero_pad = jnp.zeros((EPAD - EE,), jnp.int32)
    src_deg = jnp.concatenate([edge_index[0], sink_pad]).reshape(ROWS, CB)
    src_hop = jnp.concatenate([edge_index[0], zero_pad]).reshape(ROWS, CB)
    dst_p = jnp.concatenate([edge_index[1], sink_pad]).reshape(ROWS, CB)
    ones16 = jnp.ones((CB, 16), jnp.float32)
    zeros16 = jnp.zeros((NPAD, 16), jnp.float32)
    zeros64 = jnp.zeros((NPAD, CC), jnp.bfloat16)

    degp = _deg_kernel(src_deg, dst_p, ones16, zeros16)
    y0 = _matmul(features, W)
    g1, nrm = _scale(y0, degp)
    a1 = _hop_kernel(g1, src_hop, dst_p, zeros64)
    g2 = _mid(a1, nrm)
    a2 = _hop_kernel(g2, src_hop, dst_p, zeros64)
    probs, logits, ent = _fin(a2, nrm)
    return probs, logits, ent[0, 0]
